# Initial kernel scaffold; baseline (speedup 1.0000x reference)
import functools

import jax
import jax.numpy as jnp
from jax import lax
from jax.experimental import pallas as pl
from jax.experimental.pallas import tpu as pltpu
from jax.experimental.pallas import tpu_sc as plsc

NC = 2
NS = 16
HALF = 128


def _chunk(epw):
    for k in range(128, 0, -8):
        if epw % k == 0:
            return k
    raise ValueError(epw)


def _build_agg_wide(n, e):
    assert e % NS == 0 and n % NS == 0
    epw = e // NS
    k = _chunk(epw)
    nchunks = epw // k
    rpw = n // NS
    mesh = plsc.VectorSubcoreMesh(
        core_axis_name="c", subcore_axis_name="s", num_cores=NC, num_subcores=NS
    )

    @functools.partial(
        pl.kernel,
        out_type=(
            jax.ShapeDtypeStruct((n, HALF), jnp.float32),
            jax.ShapeDtypeStruct((n, HALF), jnp.float32),
        ),
        mesh=mesh,
        scratch_types=[
            pltpu.VMEM((k,), jnp.int32),
            pltpu.VMEM((k,), jnp.int32),
            pltpu.VMEM((k, HALF), jnp.float32),
            pltpu.VMEM_SHARED((n, HALF), jnp.float32),
            pltpu.SemaphoreType.DMA,
        ],
    )
    def kern(lo_hbm, hi_hbm, row_hbm, col_hbm, out_lo, out_hi,
             row_v, col_v, buf_v, acc_sh, sem):
        cid = lax.axis_index("c")
        sid = lax.axis_index("s")
        ebase = sid * epw
        r0 = sid * rpw

        def run(x_hbm, out_hbm):
            pltpu.sync_copy(x_hbm.at[pl.ds(r0, rpw)], acc_sh.at[pl.ds(r0, rpw)])
            plsc.subcore_barrier()

            def body(j, _):
                off = ebase + j * k
                pltpu.sync_copy(row_hbm.at[pl.ds(off, k)], row_v)
                pltpu.sync_copy(col_hbm.at[pl.ds(off, k)], col_v)
                pltpu.async_copy(x_hbm.at[row_v], buf_v, sem).wait()
                pltpu.sync_copy(buf_v, acc_sh.at[col_v], add=True)
                return 0

            lax.fori_loop(0, nchunks, body, 0)
            plsc.subcore_barrier()
            pltpu.sync_copy(acc_sh.at[pl.ds(r0, rpw)], out_hbm.at[pl.ds(r0, rpw)])

        @pl.when(cid == 0)
        def _():
            run(lo_hbm, out_lo)

        @pl.when(cid == 1)
        def _():
            run(hi_hbm, out_hi)

    return kern


def _build_agg_small(n, e):
    w = 16
    assert e % (NC * NS) == 0 and n % NS == 0
    epw = e // (NC * NS)
    k = _chunk(epw)
    nchunks = epw // k
    rpw = n // NS
    mesh = plsc.VectorSubcoreMesh(
        core_axis_name="c", subcore_axis_name="s", num_cores=NC, num_subcores=NS
    )

    @functools.partial(
        pl.kernel,
        out_type=(
            jax.ShapeDtypeStruct((n, w), jnp.float32),
            jax.ShapeDtypeStruct((n, w), jnp.float32),
        ),
        mesh=mesh,
        scratch_types=[
            pltpu.VMEM((k,), jnp.int32),
            pltpu.VMEM((k,), jnp.int32),
            pltpu.VMEM((k, w), jnp.float32),
            pltpu.VMEM_SHARED((n, w), jnp.float32),
            pltpu.SemaphoreType.DMA,
        ],
    )
    def kern(tab_hbm, row_hbm, col_hbm, p0_hbm, p1_hbm,
             row_v, col_v, buf_v, acc_sh, sem):
        cid = lax.axis_index("c")
        sid = lax.axis_index("s")
        ebase = (cid * NS + sid) * epw
        r0 = sid * rpw

        pltpu.sync_copy(tab_hbm.at[pl.ds(r0, rpw)], acc_sh.at[pl.ds(r0, rpw)])
        plsc.subcore_barrier()

        def body(j, _):
            off = ebase + j * k
            pltpu.sync_copy(row_hbm.at[pl.ds(off, k)], row_v)
            pltpu.sync_copy(col_hbm.at[pl.ds(off, k)], col_v)
            pltpu.async_copy(tab_hbm.at[row_v], buf_v, sem).wait()
            pltpu.sync_copy(buf_v, acc_sh.at[col_v], add=True)
            return 0

        lax.fori_loop(0, nchunks, body, 0)
        plsc.subcore_barrier()

        @pl.when(cid == 0)
        def _():
            pltpu.sync_copy(acc_sh.at[pl.ds(r0, rpw)], p0_hbm.at[pl.ds(r0, rpw)])

        @pl.when(cid == 1)
        def _():
            pltpu.sync_copy(acc_sh.at[pl.ds(r0, rpw)], p1_hbm.at[pl.ds(r0, rpw)])

    return kern


def kernel(x, edge_index, W0, b0, W1, b1, W2, b2):
    n, f_in = x.shape
    e = edge_index.shape[1]
    row = edge_index[0]
    col = edge_index[1]

    agg_wide = _build_agg_wide(n, e)
    agg_small = _build_agg_small(n, e)

    ones16 = jnp.ones((n, 16), jnp.float32)
    dp0, dp1 = agg_small(ones16, row, col)
    deg = dp0[:, 0] + dp1[:, 0] - 1.0
    dinv = lax.rsqrt(deg)

    xs = dinv[:, None] * (x @ W0.T)
    a_lo, a_hi = agg_wide(xs[:, :HALF], xs[:, HALF:], row, col)
    h = jnp.tanh(dinv[:, None] * jnp.concatenate([a_lo, a_hi], axis=1) + b0)

    xs = dinv[:, None] * (h @ W1.T)
    a_lo, a_hi = agg_wide(xs[:, :HALF], xs[:, HALF:], row, col)
    h = jnp.tanh(dinv[:, None] * jnp.concatenate([a_lo, a_hi], axis=1) + b1)

    s = dinv * (h @ W2.T)[:, 0]
    s16 = jnp.broadcast_to(s[:, None], (n, 16))
    p0, p1 = agg_small(s16, row, col)
    z = dinv * (p0[:, 0] + p1[:, 0] - s) + b2[0]
    return 1.0 / (1.0 + jnp.exp(-z))


# speedup vs baseline: 10.0558x; 10.0558x over previous
import functools

import jax
import jax.numpy as jnp
from jax import lax
from jax.experimental import pallas as pl
from jax.experimental.pallas import tpu as pltpu
from jax.experimental.pallas import tpu_sc as plsc

NC = 2
NS = 16
HALF = 128


def _chunk(epw):
    for k in range(128, 0, -8):
        if epw % k == 0:
            return k
    raise ValueError(epw)


def _row_copy(sid, src, dst, n):
    rpw = (n // NS) // 8 * 8
    tail = n - NS * rpw
    r0 = sid * rpw
    pltpu.sync_copy(src.at[pl.ds(r0, rpw)], dst.at[pl.ds(r0, rpw)])
    if tail:
        @pl.when(sid == NS - 1)
        def _():
            pltpu.sync_copy(src.at[pl.ds(NS * rpw, tail)],
                            dst.at[pl.ds(NS * rpw, tail)])


def _build_agg_wide(n, e):
    assert e % NS == 0 and n % NS == 0
    epw = e // NS
    k = _chunk(epw)
    nchunks = epw // k
    rpw = n // NS
    mesh = plsc.VectorSubcoreMesh(
        core_axis_name="c", subcore_axis_name="s", num_cores=NC, num_subcores=NS
    )

    @functools.partial(
        pl.kernel,
        out_type=(
            jax.ShapeDtypeStruct((n, HALF), jnp.float32),
            jax.ShapeDtypeStruct((n, HALF), jnp.float32),
        ),
        mesh=mesh,
        scratch_types=[
            pltpu.VMEM((k,), jnp.int32),
            pltpu.VMEM((k,), jnp.int32),
            pltpu.VMEM((k, HALF), jnp.float32),
            pltpu.VMEM_SHARED((n, HALF), jnp.float32),
            pltpu.SemaphoreType.DMA,
        ],
    )
    def kern(lo_hbm, hi_hbm, row_hbm, col_hbm, out_lo, out_hi,
             row_v, col_v, buf_v, acc_sh, sem):
        cid = lax.axis_index("c")
        sid = lax.axis_index("s")
        ebase = sid * epw

        def run(x_hbm, out_hbm):
            _row_copy(sid, x_hbm, acc_sh, n)
            plsc.subcore_barrier()

            def body(j, _):
                off = ebase + j * k
                pltpu.sync_copy(row_hbm.at[pl.ds(off, k)], row_v)
                pltpu.sync_copy(col_hbm.at[pl.ds(off, k)], col_v)
                pltpu.async_copy(x_hbm.at[row_v], buf_v, sem).wait()
                pltpu.sync_copy(buf_v, acc_sh.at[col_v], add=True)
                return 0

            lax.fori_loop(0, nchunks, body, 0)
            plsc.subcore_barrier()
            _row_copy(sid, acc_sh, out_hbm, n)

        @pl.when(cid == 0)
        def _():
            run(lo_hbm, out_lo)

        @pl.when(cid == 1)
        def _():
            run(hi_hbm, out_hi)

    return kern


def _build_agg_scalar(n, e):
    nw = NC * NS
    L = 16
    assert n % L == 0
    epw = -(-e // nw)
    epw_pad = -(-epw // L) * L
    assert (epw * (nw - 1)) % 8 == 0 and epw % 8 == 0
    nvec = epw // L
    rem = epw - nvec * L
    mesh = plsc.VectorSubcoreMesh(
        core_axis_name="c", subcore_axis_name="s", num_cores=NC, num_subcores=NS
    )

    @functools.partial(
        pl.kernel,
        out_type=jax.ShapeDtypeStruct((nw * n,), jnp.float32),
        mesh=mesh,
        compiler_params=pltpu.CompilerParams(needs_layout_passes=False),
        scratch_types=[
            pltpu.VMEM((n,), jnp.float32),
            pltpu.VMEM((n,), jnp.float32),
            pltpu.VMEM((epw_pad,), jnp.int32),
            pltpu.VMEM((epw_pad,), jnp.int32),
        ],
    )
    def kern(tab_hbm, row_hbm, col_hbm, out_hbm, tab_v, acc_v, row_v, col_v):
        cid = lax.axis_index("c")
        sid = lax.axis_index("s")
        wid = cid * NS + sid
        ebase = wid * epw

        zeros = jnp.zeros((L,), jnp.float32)

        def zbody(i, _):
            acc_v[pl.ds(i * L, L)] = zeros
            return 0

        lax.fori_loop(0, n // L, zbody, 0)
        pltpu.sync_copy(tab_hbm, tab_v)
        if epw_pad > epw:
            col_v[pl.ds(epw_pad - L, L)] = jnp.zeros((L,), jnp.int32)
            row_v[pl.ds(epw_pad - L, L)] = jnp.zeros((L,), jnp.int32)
        pltpu.sync_copy(row_hbm.at[pl.ds(ebase, epw)], row_v.at[pl.ds(0, epw)])
        pltpu.sync_copy(col_hbm.at[pl.ds(ebase, epw)], col_v.at[pl.ds(0, epw)])

        def body(j, _):
            idx = row_v[pl.ds(j * L, L)]
            cols = col_v[pl.ds(j * L, L)]
            vals = plsc.load_gather(tab_v, [idx])
            plsc.addupdate_scatter(acc_v, [cols], vals)
            return 0

        lax.fori_loop(0, nvec, body, 0)
        if rem:
            mask = lax.iota(jnp.int32, L) < rem
            idx = row_v[pl.ds(nvec * L, L)]
            cols = col_v[pl.ds(nvec * L, L)]
            vals = plsc.load_gather(tab_v, [idx], mask=mask)
            plsc.addupdate_scatter(acc_v, [cols], vals, mask=mask)
        pltpu.sync_copy(acc_v, out_hbm.at[pl.ds(wid * n, n)])

    return kern


def kernel(x, edge_index, W0, b0, W1, b1, W2, b2):
    n, f_in = x.shape
    e = edge_index.shape[1]
    row = edge_index[0]
    col = edge_index[1]

    agg_wide = _build_agg_wide(n, e)
    agg_scalar = _build_agg_scalar(n, e)

    dp = agg_scalar(jnp.ones((n,), jnp.float32), row, col)
    deg = dp.reshape(NC * NS, n).sum(axis=0) + 1.0
    dinv = lax.rsqrt(deg)

    xs = dinv[:, None] * (x @ W0.T)
    a_lo, a_hi = agg_wide(xs[:, :HALF], xs[:, HALF:], row, col)
    h = jnp.tanh(dinv[:, None] * jnp.concatenate([a_lo, a_hi], axis=1) + b0)

    xs = dinv[:, None] * (h @ W1.T)
    a_lo, a_hi = agg_wide(xs[:, :HALF], xs[:, HALF:], row, col)
    h = jnp.tanh(dinv[:, None] * jnp.concatenate([a_lo, a_hi], axis=1) + b1)

    s = dinv * (h @ W2.T)[:, 0]
    sp = agg_scalar(s, row, col)
    z = dinv * (sp.reshape(NC * NS, n).sum(axis=0) + s) + b2[0]
    return 1.0 / (1.0 + jnp.exp(-z))


# speedup vs baseline: 18.5817x; 1.8479x over previous
import functools

import jax
import jax.numpy as jnp
from jax import lax
from jax.experimental import pallas as pl
from jax.experimental.pallas import tpu as pltpu
from jax.experimental.pallas import tpu_sc as plsc

NC = 2
NS = 16
HALF = 128


def _chunk(epw):
    for k in range(128, 0, -8):
        if epw % k == 0:
            return k
    raise ValueError(epw)


def _row_copy(sid, src, dst, n):
    rpw = (n // NS) // 8 * 8
    tail = n - NS * rpw
    r0 = sid * rpw
    pltpu.sync_copy(src.at[pl.ds(r0, rpw)], dst.at[pl.ds(r0, rpw)])
    if tail:
        @pl.when(sid == NS - 1)
        def _():
            pltpu.sync_copy(src.at[pl.ds(NS * rpw, tail)],
                            dst.at[pl.ds(NS * rpw, tail)])


def _build_agg_wide(n, e):
    assert e % NS == 0 and n % NS == 0
    epw = e // NS
    k = 96
    nchunks = epw // k
    tail = epw - nchunks * k
    assert tail % 8 == 0 and nchunks >= 6
    mesh = plsc.VectorSubcoreMesh(
        core_axis_name="c", subcore_axis_name="s", num_cores=NC, num_subcores=NS
    )

    idx_t = pltpu.VMEM((k,), jnp.int32)
    buf_t = pltpu.VMEM((k, HALF), jnp.float32)
    scratch = ([idx_t] * 3 + [idx_t] * 3 + [buf_t] * 3
               + ([pltpu.VMEM((tail,), jnp.int32)] * 2
                  + [pltpu.VMEM((tail, HALF), jnp.float32)] if tail else [])
               + [pltpu.VMEM_SHARED((n, HALF), jnp.float32)]
               + [pltpu.SemaphoreType.DMA] * (10 if tail else 9))

    @functools.partial(
        pl.kernel,
        out_type=(
            jax.ShapeDtypeStruct((n, HALF), jnp.float32),
            jax.ShapeDtypeStruct((n, HALF), jnp.float32),
        ),
        mesh=mesh,
        scratch_types=scratch,
    )
    def kern(lo_hbm, hi_hbm, row_hbm, col_hbm, out_lo, out_hi, *scr):
        rows = scr[0:3]
        cols = scr[3:6]
        bufs = scr[6:9]
        if tail:
            trow, tcol, tbuf = scr[9:12]
            acc_sh = scr[12]
            sems = scr[13:]
        else:
            acc_sh = scr[9]
            sems = scr[10:]
        semi = sems[0:3]
        semg = sems[3:6]
        sems_ = sems[6:9]

        cid = lax.axis_index("c")
        sid = lax.axis_index("s")
        ebase = sid * epw

        def run(x_hbm, out_hbm):
            def issue_i(j, r):
                off = ebase + j * k
                pltpu.async_copy(row_hbm.at[pl.ds(off, k)], rows[r], semi[r])
                pltpu.async_copy(col_hbm.at[pl.ds(off, k)], cols[r], semi[r])

            def wait_i(r):
                pltpu.make_async_copy(row_hbm.at[pl.ds(0, k)], rows[r], semi[r]).wait()
                pltpu.make_async_copy(col_hbm.at[pl.ds(0, k)], cols[r], semi[r]).wait()

            def issue_g(r):
                pltpu.async_copy(x_hbm.at[rows[r]], bufs[r], semg[r])

            def wait_g(r):
                pltpu.make_async_copy(x_hbm.at[rows[r]], bufs[r], semg[r]).wait()

            def issue_s(r):
                pltpu.async_copy(bufs[r], acc_sh.at[cols[r]], sems_[r], add=True)

            def wait_s(r):
                pltpu.make_async_copy(bufs[r], acc_sh.at[cols[r]], sems_[r]).wait()

            def body(j, r, first=False, has_i=True, has_g=True):
                if not first:
                    wait_s((r + 2) % 3)
                if has_i:
                    issue_i(j + 2, (r + 2) % 3)
                wait_g(r)
                issue_s(r)
                if has_g:
                    wait_i((r + 1) % 3)
                    issue_g((r + 1) % 3)

            _row_copy(sid, x_hbm, acc_sh, n)
            plsc.subcore_barrier()

            issue_i(0, 0)
            issue_i(1, 1)
            wait_i(0)
            issue_g(0)
            body(0, 0, first=True)
            body(1, 1)

            steady = ((nchunks - 4) // 3) * 3
            if steady > 0:
                def loop_body(u, _):
                    j0 = 2 + u * 3
                    for p in range(3):
                        body(j0 + p, (2 + p) % 3)
                    return 0

                lax.fori_loop(0, steady // 3, loop_body, 0)

            for j in range(2 + steady, nchunks):
                body(j, j % 3, has_i=(j + 2 < nchunks), has_g=(j + 1 < nchunks))
            wait_s((nchunks - 1) % 3)

            if tail:
                off = ebase + nchunks * k
                pltpu.sync_copy(row_hbm.at[pl.ds(off, tail)], trow)
                pltpu.sync_copy(col_hbm.at[pl.ds(off, tail)], tcol)
                pltpu.async_copy(x_hbm.at[trow], tbuf, sems[9]).wait()
                pltpu.sync_copy(tbuf, acc_sh.at[tcol], add=True)

            plsc.subcore_barrier()
            _row_copy(sid, acc_sh, out_hbm, n)

        @pl.when(cid == 0)
        def _():
            run(lo_hbm, out_lo)

        @pl.when(cid == 1)
        def _():
            run(hi_hbm, out_hi)

    return kern


def _build_agg_scalar(n, e):
    nw = NC * NS
    L = 16
    assert n % L == 0
    epw = -(-e // nw)
    epw_pad = -(-epw // L) * L
    assert (epw * (nw - 1)) % 8 == 0 and epw % 8 == 0
    nvec = epw // L
    rem = epw - nvec * L
    mesh = plsc.VectorSubcoreMesh(
        core_axis_name="c", subcore_axis_name="s", num_cores=NC, num_subcores=NS
    )

    @functools.partial(
        pl.kernel,
        out_type=jax.ShapeDtypeStruct((nw * n,), jnp.float32),
        mesh=mesh,
        compiler_params=pltpu.CompilerParams(needs_layout_passes=False),
        scratch_types=[
            pltpu.VMEM((n,), jnp.float32),
            pltpu.VMEM((n,), jnp.float32),
            pltpu.VMEM((epw_pad,), jnp.int32),
            pltpu.VMEM((epw_pad,), jnp.int32),
        ],
    )
    def kern(tab_hbm, row_hbm, col_hbm, out_hbm, tab_v, acc_v, row_v, col_v):
        cid = lax.axis_index("c")
        sid = lax.axis_index("s")
        wid = cid * NS + sid
        ebase = wid * epw

        zeros = jnp.zeros((L,), jnp.float32)

        def zbody(i, _):
            acc_v[pl.ds(i * L, L)] = zeros
            return 0

        lax.fori_loop(0, n // L, zbody, 0)
        pltpu.sync_copy(tab_hbm, tab_v)
        if epw_pad > epw:
            col_v[pl.ds(epw_pad - L, L)] = jnp.zeros((L,), jnp.int32)
            row_v[pl.ds(epw_pad - L, L)] = jnp.zeros((L,), jnp.int32)
        pltpu.sync_copy(row_hbm.at[pl.ds(ebase, epw)], row_v.at[pl.ds(0, epw)])
        pltpu.sync_copy(col_hbm.at[pl.ds(ebase, epw)], col_v.at[pl.ds(0, epw)])

        def body(j, _):
            idx = row_v[pl.ds(j * L, L)]
            cols = col_v[pl.ds(j * L, L)]
            vals = plsc.load_gather(tab_v, [idx])
            plsc.addupdate_scatter(acc_v, [cols], vals)
            return 0

        lax.fori_loop(0, nvec, body, 0)
        if rem:
            mask = lax.iota(jnp.int32, L) < rem
            idx = row_v[pl.ds(nvec * L, L)]
            cols = col_v[pl.ds(nvec * L, L)]
            vals = plsc.load_gather(tab_v, [idx], mask=mask)
            plsc.addupdate_scatter(acc_v, [cols], vals, mask=mask)
        pltpu.sync_copy(acc_v, out_hbm.at[pl.ds(wid * n, n)])

    return kern


def _mm_first(x_ref, w_ref, dp_ref, lo_ref, hi_ref, dinv_ref):
    deg = jnp.sum(dp_ref[...], axis=0) + 1.0
    dinv = lax.rsqrt(deg)
    dinv_ref[...] = dinv
    xl = lax.dot_general(x_ref[...], w_ref[...], (((1,), (1,)), ((), ())),
                         preferred_element_type=jnp.float32)
    xs = xl * dinv[:, None]
    lo_ref[...] = xs[:, :HALF]
    hi_ref[...] = xs[:, HALF:]


def _mm_mid(lo_ref, hi_ref, dinv_ref, b_ref, w_ref, olo_ref, ohi_ref):
    dinv = dinv_ref[...][:, None]
    b = b_ref[...]
    h_lo = jnp.tanh(dinv * lo_ref[...] + b[:, :HALF])
    h_hi = jnp.tanh(dinv * hi_ref[...] + b[:, HALF:])
    w = w_ref[...]
    xl = (lax.dot_general(h_lo, w[:, :HALF], (((1,), (1,)), ((), ())),
                          preferred_element_type=jnp.float32)
          + lax.dot_general(h_hi, w[:, HALF:], (((1,), (1,)), ((), ())),
                            preferred_element_type=jnp.float32))
    xs = xl * dinv
    olo_ref[...] = xs[:, :HALF]
    ohi_ref[...] = xs[:, HALF:]


def _mm_last(lo_ref, hi_ref, dinv_ref, b_ref, w2_ref, s_ref):
    dinv = dinv_ref[...][:, None]
    b = b_ref[...]
    h_lo = jnp.tanh(dinv * lo_ref[...] + b[:, :HALF])
    h_hi = jnp.tanh(dinv * hi_ref[...] + b[:, HALF:])
    w2 = w2_ref[...]
    s = (lax.dot_general(h_lo, w2[:, :HALF], (((1,), (1,)), ((), ())),
                         preferred_element_type=jnp.float32)
         + lax.dot_general(h_hi, w2[:, HALF:], (((1,), (1,)), ((), ())),
                           preferred_element_type=jnp.float32))
    s_ref[...] = (s * dinv)[:, 0]


def _mm_fin(sp_ref, s_ref, dinv_ref, b2_ref, o_ref):
    agg = jnp.sum(sp_ref[...], axis=0) + s_ref[...]
    z = dinv_ref[...] * agg + b2_ref[...]
    o_ref[...] = 1.0 / (1.0 + jnp.exp(-z))


def kernel(x, edge_index, W0, b0, W1, b1, W2, b2):
    n, f_in = x.shape
    d = W0.shape[0]
    e = edge_index.shape[1]
    nw = NC * NS
    row = edge_index[0]
    col = edge_index[1]

    agg_wide = _build_agg_wide(n, e)
    agg_scalar = _build_agg_scalar(n, e)
    f32 = jnp.float32

    dp = agg_scalar(jnp.ones((n,), f32), row, col).reshape(nw, n)

    xs_lo, xs_hi, dinv = pl.pallas_call(
        _mm_first,
        out_shape=(jax.ShapeDtypeStruct((n, HALF), f32),
                   jax.ShapeDtypeStruct((n, HALF), f32),
                   jax.ShapeDtypeStruct((n,), f32)),
    )(x, W0, dp)
    a_lo, a_hi = agg_wide(xs_lo, xs_hi, row, col)

    xs_lo, xs_hi = pl.pallas_call(
        _mm_mid,
        out_shape=(jax.ShapeDtypeStruct((n, HALF), f32),
                   jax.ShapeDtypeStruct((n, HALF), f32)),
    )(a_lo, a_hi, dinv, b0.reshape(1, d), W1)
    a_lo, a_hi = agg_wide(xs_lo, xs_hi, row, col)

    s = pl.pallas_call(
        _mm_last,
        out_shape=jax.ShapeDtypeStruct((n,), f32),
    )(a_lo, a_hi, dinv, b1.reshape(1, d), W2)
    sp = agg_scalar(s, row, col).reshape(nw, n)
    return pl.pallas_call(
        _mm_fin,
        out_shape=jax.ShapeDtypeStruct((n,), f32),
    )(sp, s, dinv, b2)
